# Initial kernel scaffold; baseline (speedup 1.0000x reference)
import jax
import jax.numpy as jnp
from jax.experimental import pallas as pl


def kernel(X, edge_index, edge_weight, H, W_xz, b_xz, W_hz, b_hz, W_xr, b_xr, W_hr, b_hr, W_xh, b_xh, W_hh, b_hh):
    raise NotImplementedError("write your pallas kernel here")


# speedup vs baseline: 1.1598x; 1.1598x over previous
import functools

import jax
import jax.numpy as jnp
from jax import lax
from jax.experimental import pallas as pl
from jax.experimental.pallas import tpu as pltpu
from jax.experimental.pallas import tpu_sc as plsc

N = 10000
NP = 10240
NH = NP // 2
NA = NH + 8
E = 160000
C = 256
CH = 128
NT = 16
EPT = E // NT
NCH = EPT // 16
RPT = NP // NT
APT = NH // NT

_i32 = jnp.int32
_f32 = jnp.float32


def _full16(v):
    return jnp.full((16,), v, _i32)


_MESH = plsc.VectorSubcoreMesh(core_axis_name="c", subcore_axis_name="s")
_SC_PARAMS = pltpu.CompilerParams(needs_layout_passes=False)


EPW = E // 32


@functools.partial(
    pl.kernel,
    mesh=_MESH,
    compiler_params=_SC_PARAMS,
    out_type=jax.ShapeDtypeStruct((32, 1, NP), _f32),
    scratch_types=[
        pltpu.VMEM((NP,), _f32),
        pltpu.VMEM((EPW,), _i32),
        pltpu.VMEM((EPW,), _f32),
    ],
)
def _degpart(rowW, wW, zN, degP, accf, rowf, wf):
    c = lax.axis_index("c")
    s = lax.axis_index("s")
    wid = s * 2 + c
    pltpu.sync_copy(rowW.at[wid, 0], rowf)
    pltpu.sync_copy(wW.at[wid, 0], wf)
    pltpu.sync_copy(zN.at[0], accf)
    lane0 = lax.iota(_i32, 16) == 0

    def body(e, carry):
        rsp = plsc.load_gather(rowf, [_full16(e)])
        wsp = plsc.load_gather(wf, [_full16(e)])
        plsc.addupdate_scatter(accf, [rsp], wsp, mask=lane0)
        return carry

    lax.fori_loop(0, EPW, body, 0)
    pltpu.sync_copy(accf, degP.at[wid, 0])


def _dis_body(p_ref, o_ref):
    deg = jnp.sum(p_ref[...], axis=0)
    deg_safe = jnp.where(deg > 0, deg, 1.0)
    o_ref[...] = jnp.where(deg > 0, lax.rsqrt(deg_safe), 0.0)[None, :]


def _dis(degP):
    return pl.pallas_call(
        _dis_body,
        out_shape=jax.ShapeDtypeStruct((1, NP), _f32),
    )(degP)


@functools.partial(
    pl.kernel,
    mesh=_MESH,
    compiler_params=_SC_PARAMS,
    out_type=jax.ShapeDtypeStruct((NT, 1, EPT), _f32),
    scratch_types=[
        pltpu.VMEM((EPT,), _i32),
        pltpu.VMEM((EPT,), _i32),
        pltpu.VMEM((EPT,), _f32),
        pltpu.VMEM((NP,), _f32),
        pltpu.VMEM((EPT,), _f32),
    ],
)
def _norm(rowTf, colTf, wTf, dis_r, normTf, rowf, colf, wf, dfull, nout):
    c = lax.axis_index("c")
    s = lax.axis_index("s")

    @pl.when(c == 0)
    def _():
        pltpu.sync_copy(rowTf.at[s, 0], rowf)
        pltpu.sync_copy(colTf.at[s, 0], colf)
        pltpu.sync_copy(wTf.at[s, 0], wf)
        pltpu.sync_copy(dis_r.at[0], dfull)
        iota = lax.iota(_i32, 16)

        def body(jc, carry):
            e0 = jc * 16
            rr = rowf[pl.ds(e0, 16)]
            cc = colf[pl.ds(e0, 16)]
            ww = wf[pl.ds(e0, 16)]
            a = plsc.load_gather(dfull, [rr])
            b = plsc.load_gather(dfull, [cc])
            nv = -(a * ww) * b
            plsc.store_scatter(nout, [e0 + iota], nv)
            return carry

        lax.fori_loop(0, NCH, body, 0)
        pltpu.sync_copy(nout, normTf.at[s, 0])


_HLF = jax.ShapeDtypeStruct((NP, CH), _f32)


@functools.partial(
    pl.kernel,
    mesh=_MESH,
    compiler_params=_SC_PARAMS,
    out_type=(_HLF,) * 4,
    scratch_types=[
        pltpu.VMEM_SHARED((NA, CH), _f32),
        pltpu.VMEM((EPT,), _i32),
        pltpu.VMEM((EPT,), _i32),
        pltpu.VMEM((EPT,), _f32),
        pltpu.VMEM((16, CH), _f32),
        pltpu.VMEM((16,), _i32),
        pltpu.SemaphoreType.DMA,
    ],
)
def _chain(vh0, vh1, rowTf, colTf, normTf, zacc,
           l1h0, l1h1, l2h0, l2h1,
           acc, rowf, colf, normf, gbuf, idxbuf, sem_g):
    c = lax.axis_index("c")
    s = lax.axis_index("s")
    pltpu.sync_copy(rowTf.at[s, 0], rowf)
    pltpu.sync_copy(colTf.at[s, 0], colf)
    pltpu.sync_copy(normTf.at[s, 0], normf)
    myacc = pl.ds(s * APT, APT)

    def one_pass(src, dst, p):
        base = NH * p
        pltpu.sync_copy(zacc.at[myacc], acc.at[myacc])

        @pl.when(s == 0)
        def _():
            pltpu.sync_copy(zacc.at[pl.ds(NH, 8)], acc.at[pl.ds(NH, 8)])

        plsc.subcore_barrier()

        def body(jc, carry):
            e0 = jc * 16
            pltpu.async_copy(src.at[rowf.at[pl.ds(e0, 16)]], gbuf,
                             sem_g).wait()
            local = colf[pl.ds(e0, 16)] - base
            ok = (local >= 0) & (local < NH)
            idxbuf[...] = jnp.where(ok, local, NH)
            for j in range(16):
                spl = plsc.load_gather(normf, [_full16(e0 + j)])
                for h in range(CH // 16):
                    gbuf[j, pl.ds(16 * h, 16)] = (
                        gbuf[j, pl.ds(16 * h, 16)] * spl)
            pltpu.sync_copy(gbuf, acc.at[idxbuf], add=True)
            return carry

        lax.fori_loop(0, NCH, body, 0)
        plsc.subcore_barrier()
        pltpu.sync_copy(acc.at[pl.ds(s * APT, APT)],
                        dst.at[pl.ds(base + s * APT, APT)])
        plsc.subcore_barrier()

    def chain_for(vh, l1h, l2h):
        one_pass(vh, l1h, 0)
        one_pass(vh, l1h, 1)
        one_pass(l1h, l2h, 0)
        one_pass(l1h, l2h, 1)

    @pl.when(c == 0)
    def _():
        chain_for(vh0, l1h0, l2h0)

    @pl.when(c == 1)
    def _():
        chain_for(vh1, l1h1, l2h1)


_BM = 1024
_GRID = NP // _BM


def _dots(W_ref, ins):
    acc = jnp.dot(ins[0][...], W_ref[0], preferred_element_type=_f32)
    for k in range(1, len(ins)):
        acc = acc + jnp.dot(ins[k][...], W_ref[k], preferred_element_type=_f32)
    return acc


def _mm1_body(W_ref, *refs):
    ins, o_ref = refs[:6], refs[6]
    o_ref[...] = _dots(W_ref, ins)


def _mm2_body(W_ref, *refs):
    ins = refs[:6]
    a1_ref, bzr_ref = refs[6], refs[7]
    z_ref, g0_ref, g1_ref = refs[8:]
    acc = _dots(W_ref, ins)
    gz = jax.nn.sigmoid(acc + a1_ref[...] + bzr_ref[...])
    z = gz[:, :C]
    r = gz[:, C:]
    z_ref[...] = z
    g0_ref[...] = ins[0][...] * r[:, :CH]
    g1_ref[...] = ins[1][...] * r[:, CH:]


def _mm3_body(W_ref, *refs):
    ins = refs[:6]
    a1h_ref, bh_ref, z_ref, h0, h1, o_ref = refs[6:]
    acc = _dots(W_ref, ins)
    ht = jnp.tanh(acc + a1h_ref[...] + bh_ref[...])
    z = z_ref[...]
    hcat = jnp.concatenate([h0[...], h1[...]], axis=1)
    o_ref[...] = z * hcat + (1.0 - z) * ht


def _hspec():
    return pl.BlockSpec((_BM, CH), lambda i: (i, 0))


def _mm1(Wxs, ins):
    return pl.pallas_call(
        _mm1_body,
        grid=(_GRID,),
        in_specs=[pl.BlockSpec((6, CH, 3 * C), lambda i: (0, 0, 0))]
        + [_hspec() for _ in range(6)],
        out_specs=pl.BlockSpec((_BM, 3 * C), lambda i: (i, 0)),
        out_shape=jax.ShapeDtypeStruct((NP, 3 * C), _f32),
    )(Wxs, *ins)


def _mm2(Whs, ins, A1, bzr):
    return pl.pallas_call(
        _mm2_body,
        grid=(_GRID,),
        in_specs=[pl.BlockSpec((6, CH, 2 * C), lambda i: (0, 0, 0))]
        + [_hspec() for _ in range(6)]
        + [pl.BlockSpec((_BM, 2 * C), lambda i: (i, 0)),
           pl.BlockSpec((1, 2 * C), lambda i: (0, 0))],
        out_specs=(pl.BlockSpec((_BM, C), lambda i: (i, 0)),
                   _hspec(), _hspec()),
        out_shape=(jax.ShapeDtypeStruct((NP, C), _f32),
                   jax.ShapeDtypeStruct((NP, CH), _f32),
                   jax.ShapeDtypeStruct((NP, CH), _f32)),
    )(Whs, *ins, A1, bzr)


def _mm3(Wgs, ins, A1, bh, Z, h0, h1):
    return pl.pallas_call(
        _mm3_body,
        grid=(_GRID,),
        in_specs=[pl.BlockSpec((6, CH, C), lambda i: (0, 0, 0))]
        + [_hspec() for _ in range(6)]
        + [pl.BlockSpec((_BM, C), lambda i: (i, 2)),
           pl.BlockSpec((1, C), lambda i: (0, 0)),
           pl.BlockSpec((_BM, C), lambda i: (i, 0)),
           _hspec(), _hspec()],
        out_specs=pl.BlockSpec((_BM, C), lambda i: (i, 0)),
        out_shape=jax.ShapeDtypeStruct((NP, C), _f32),
    )(Wgs, *ins, A1, bh, Z, h0, h1)


def _eff(W):
    return jnp.stack([W[0] - W[2], W[1], 2.0 * W[2]])


def _split6(Ws):
    return jnp.concatenate(
        [jnp.stack([Ws[k, :CH], Ws[k, CH:]]) for k in range(3)])


def kernel(X, edge_index, edge_weight, H,
           W_xz, b_xz, W_hz, b_hz,
           W_xr, b_xr, W_hr, b_hr,
           W_xh, b_xh, W_hh, b_hh):
    row = edge_index[0].astype(_i32)
    col = edge_index[1].astype(_i32)
    rowTf = row.reshape(NT, 1, EPT)
    colTf = col.reshape(NT, 1, EPT)
    wTf = edge_weight.reshape(NT, 1, EPT)
    rowW = row.reshape(32, 1, EPW)
    wW = edge_weight.reshape(32, 1, EPW)

    pad = ((0, NP - N), (0, 0))
    Xp = jnp.pad(X, pad)
    Hp = jnp.pad(H, pad)
    X0, X1 = Xp[:, :CH], Xp[:, CH:]
    H0, H1 = Hp[:, :CH], Hp[:, CH:]
    zN = jnp.zeros((1, NP), _f32)
    zacc = jnp.zeros((NA, CH), _f32)

    Wxs = _split6(jnp.concatenate([_eff(W_xz), _eff(W_xr), _eff(W_xh)],
                                  axis=2))
    Whs = _split6(jnp.concatenate([_eff(W_hz), _eff(W_hr)], axis=2))
    Wgs = _split6(_eff(W_hh))
    bzr = jnp.concatenate([b_xz + b_hz, b_xr + b_hr])[None, :]
    bh = (b_xh + b_hh)[None, :]

    degP = _degpart(rowW, wW, zN)
    dis_r = _dis(degP.reshape(32, NP))
    normTf = _norm(rowTf, colTf, wTf, dis_r)

    L1x0, L1x1, L2x0, L2x1 = _chain(X0, X1, rowTf, colTf, normTf, zacc)
    A1 = _mm1(Wxs, (X0, X1, L1x0, L1x1, L2x0, L2x1))

    L1h0, L1h1, L2h0, L2h1 = _chain(H0, H1, rowTf, colTf, normTf, zacc)
    Z, G0, G1 = _mm2(Whs, (H0, H1, L1h0, L1h1, L2h0, L2h1), A1, bzr)

    L1g0, L1g1, L2g0, L2g1 = _chain(G0, G1, rowTf, colTf, normTf, zacc)
    out = _mm3(Wgs, (G0, G1, L1g0, L1g1, L2g0, L2g1), A1, bh, Z, H0, H1)
    return out[:N]


# speedup vs baseline: 3.0533x; 2.6326x over previous
import functools

import jax
import jax.numpy as jnp
from jax import lax
from jax.experimental import pallas as pl
from jax.experimental.pallas import tpu as pltpu
from jax.experimental.pallas import tpu_sc as plsc

N = 10000
NP = 10240
NH = NP // 2
NA = NH + 8
E = 160000
C = 256
CH = 128
NT = 16
EPT = E // NT
NCH = EPT // 16
RPT = NP // NT
APT = NH // NT

_i32 = jnp.int32
_f32 = jnp.float32


def _full16(v):
    return jnp.full((16,), v, _i32)


_MESH = plsc.VectorSubcoreMesh(core_axis_name="c", subcore_axis_name="s")
_SC_PARAMS = pltpu.CompilerParams(needs_layout_passes=False)


EPW = E // 32


@functools.partial(
    pl.kernel,
    mesh=_MESH,
    compiler_params=_SC_PARAMS,
    out_type=jax.ShapeDtypeStruct((32, 1, NP), _f32),
    scratch_types=[
        pltpu.VMEM((NP,), _f32),
        pltpu.VMEM((EPW,), _i32),
        pltpu.VMEM((EPW,), _f32),
    ],
)
def _degpart(rowW, wW, zN, degP, accf, rowf, wf):
    c = lax.axis_index("c")
    s = lax.axis_index("s")
    wid = s * 2 + c
    pltpu.sync_copy(rowW.at[wid, 0], rowf)
    pltpu.sync_copy(wW.at[wid, 0], wf)
    pltpu.sync_copy(zN.at[0], accf)
    lane0 = lax.iota(_i32, 16) == 0

    def body(e, carry):
        rsp = plsc.load_gather(rowf, [_full16(e)])
        wsp = plsc.load_gather(wf, [_full16(e)])
        plsc.addupdate_scatter(accf, [rsp], wsp, mask=lane0)
        return carry

    lax.fori_loop(0, EPW, body, 0)
    pltpu.sync_copy(accf, degP.at[wid, 0])


def _dis_body(p_ref, o_ref):
    deg = jnp.sum(p_ref[...], axis=0)
    deg_safe = jnp.where(deg > 0, deg, 1.0)
    o_ref[...] = jnp.where(deg > 0, lax.rsqrt(deg_safe), 0.0)[None, :]


def _dis(degP):
    return pl.pallas_call(
        _dis_body,
        out_shape=jax.ShapeDtypeStruct((1, NP), _f32),
    )(degP)


@functools.partial(
    pl.kernel,
    mesh=_MESH,
    compiler_params=_SC_PARAMS,
    out_type=jax.ShapeDtypeStruct((NT, 1, EPT), _f32),
    scratch_types=[
        pltpu.VMEM((EPT,), _i32),
        pltpu.VMEM((EPT,), _i32),
        pltpu.VMEM((EPT,), _f32),
        pltpu.VMEM((NP,), _f32),
        pltpu.VMEM((EPT,), _f32),
    ],
)
def _norm(rowTf, colTf, wTf, dis_r, normTf, rowf, colf, wf, dfull, nout):
    c = lax.axis_index("c")
    s = lax.axis_index("s")

    @pl.when(c == 0)
    def _():
        pltpu.sync_copy(rowTf.at[s, 0], rowf)
        pltpu.sync_copy(colTf.at[s, 0], colf)
        pltpu.sync_copy(wTf.at[s, 0], wf)
        pltpu.sync_copy(dis_r.at[0], dfull)
        iota = lax.iota(_i32, 16)

        def body(jc, carry):
            e0 = jc * 16
            rr = rowf[pl.ds(e0, 16)]
            cc = colf[pl.ds(e0, 16)]
            ww = wf[pl.ds(e0, 16)]
            a = plsc.load_gather(dfull, [rr])
            b = plsc.load_gather(dfull, [cc])
            nv = -(a * ww) * b
            plsc.store_scatter(nout, [e0 + iota], nv)
            return carry

        lax.fori_loop(0, NCH, body, 0)
        pltpu.sync_copy(nout, normTf.at[s, 0])


def _scale_chunk(gbuf, normf, colf, idxbuf, e0, base, clamp):
    if clamp:
        local = colf[pl.ds(e0, 16)] - base
        ok = (local >= 0) & (local < NH)
        idxbuf[...] = jnp.where(ok, local, NH)
    else:
        idxbuf[...] = colf[pl.ds(e0, 16)]
    for j in range(16):
        spl = plsc.load_gather(normf, [_full16(e0 + j)])
        for h in range(CH // 16):
            gbuf[j, pl.ds(16 * h, 16)] = gbuf[j, pl.ds(16 * h, 16)] * spl


def _lap_pass(src, dst, acc, rowf, colf, normf,
              ga, gb, ia, ib, sga, sgb, ssa, ssb,
              zacc, s, base, n_acc_rows, clamp):
    apt = n_acc_rows // NT
    pltpu.sync_copy(zacc.at[pl.ds(s * apt, apt)], acc.at[pl.ds(s * apt, apt)])
    if clamp:
        @pl.when(s == 0)
        def _():
            pltpu.sync_copy(zacc.at[pl.ds(NH, 8)], acc.at[pl.ds(NH, 8)])
    plsc.subcore_barrier()

    pltpu.async_copy(src.at[rowf.at[pl.ds(0, 16)]], ga, sga)

    def body(j, carry):
        ea = j * 32
        eb = ea + 16
        pltpu.async_copy(src.at[rowf.at[pl.ds(eb, 16)]], gb, sgb)
        pltpu.make_async_copy(src.at[rowf.at[pl.ds(ea, 16)]], ga, sga).wait()
        _scale_chunk(ga, normf, colf, ia, ea, base, clamp)
        pltpu.async_copy(ga, acc.at[ia], ssa, add=True)
        pltpu.make_async_copy(src.at[rowf.at[pl.ds(eb, 16)]], gb, sgb).wait()
        _scale_chunk(gb, normf, colf, ib, eb, base, clamp)
        pltpu.async_copy(gb, acc.at[ib], ssb, add=True)
        pltpu.make_async_copy(ga, acc.at[ia], ssa).wait()
        pltpu.async_copy(src.at[rowf.at[pl.ds(ea + 32, 16)]], ga, sga)
        pltpu.make_async_copy(gb, acc.at[ib], ssb).wait()
        return carry

    lax.fori_loop(0, (NCH - 1) // 2, body, 0)
    e_last = (NCH - 1) * 16
    pltpu.make_async_copy(src.at[rowf.at[pl.ds(e_last, 16)]], ga, sga).wait()
    _scale_chunk(ga, normf, colf, ia, e_last, base, clamp)
    pltpu.async_copy(ga, acc.at[ia], ssa, add=True)
    pltpu.make_async_copy(ga, acc.at[ia], ssa).wait()
    plsc.subcore_barrier()
    pltpu.sync_copy(acc.at[pl.ds(s * apt, apt)],
                    dst.at[pl.ds(base + s * apt, apt)])
    plsc.subcore_barrier()


_SC_EDGE_SCRATCH = [
    pltpu.VMEM((EPT,), _i32),
    pltpu.VMEM((EPT,), _i32),
    pltpu.VMEM((EPT,), _f32),
    pltpu.VMEM((16, CH), _f32),
    pltpu.VMEM((16, CH), _f32),
    pltpu.VMEM((16,), _i32),
    pltpu.VMEM((16,), _i32),
    pltpu.SemaphoreType.DMA,
    pltpu.SemaphoreType.DMA,
    pltpu.SemaphoreType.DMA,
    pltpu.SemaphoreType.DMA,
]

_STK = jax.ShapeDtypeStruct((2, NP, CH), _f32)


@functools.partial(
    pl.kernel,
    mesh=_MESH,
    compiler_params=_SC_PARAMS,
    out_type=(_STK,) * 4,
    scratch_types=[pltpu.VMEM_SHARED((NP, CH), _f32)] + _SC_EDGE_SCRATCH,
)
def _chainxh(xs, hs, rowTf, colTf, normTf, zacc,
             l1x, l2x, l1h, l2h,
             acc, rowf, colf, normf, ga, gb, ia, ib, sga, sgb, ssa, ssb):
    c = lax.axis_index("c")
    s = lax.axis_index("s")
    pltpu.sync_copy(rowTf.at[s, 0], rowf)
    pltpu.sync_copy(colTf.at[s, 0], colf)
    pltpu.sync_copy(normTf.at[s, 0], normf)

    def run(src, dst):
        _lap_pass(src.at[c], dst.at[c], acc, rowf, colf, normf,
                  ga, gb, ia, ib, sga, sgb, ssa, ssb,
                  zacc, s, 0, NP, False)

    run(xs, l1x)
    run(l1x, l2x)
    run(hs, l1h)
    run(l1h, l2h)


@functools.partial(
    pl.kernel,
    mesh=_MESH,
    compiler_params=_SC_PARAMS,
    out_type=(_STK,) * 2,
    scratch_types=[pltpu.VMEM_SHARED((NA, CH), _f32)] + _SC_EDGE_SCRATCH,
)
def _chaing(gs, rowTf, colTf, normTf, zacc,
            l1g, l2g,
            acc, rowf, colf, normf, ga, gb, ia, ib, sga, sgb, ssa, ssb):
    c = lax.axis_index("c")
    s = lax.axis_index("s")
    pltpu.sync_copy(rowTf.at[s, 0], rowf)
    pltpu.sync_copy(colTf.at[s, 0], colf)
    pltpu.sync_copy(normTf.at[s, 0], normf)

    def run(src, dst):
        for p in range(2):
            _lap_pass(src.at[c], dst.at[c], acc, rowf, colf, normf,
                      ga, gb, ia, ib, sga, sgb, ssa, ssb,
                      zacc, s, NH * p, NH, True)

    run(gs, l1g)
    run(l1g, l2g)


_BM = 1024
_GRID = NP // _BM


def _dots(W_ref, ins):
    acc = jnp.dot(ins[0][0], W_ref[0], preferred_element_type=_f32)
    for k in range(1, len(ins)):
        acc = acc + jnp.dot(ins[k][0], W_ref[k], preferred_element_type=_f32)
    return acc


def _mm1_body(W_ref, *refs):
    ins, o_ref = refs[:6], refs[6]
    o_ref[...] = _dots(W_ref, ins)


def _mm2_body(W_ref, *refs):
    ins = refs[:6]
    a1_ref, bzr_ref = refs[6], refs[7]
    z_ref, g_ref = refs[8], refs[9]
    acc = _dots(W_ref, ins)
    gz = jax.nn.sigmoid(acc + a1_ref[...] + bzr_ref[...])
    z = gz[:, :C]
    r = gz[:, C:]
    z_ref[...] = z
    g_ref[0] = ins[0][0] * r[:, :CH]
    g_ref[1] = ins[1][0] * r[:, CH:]


def _mm3_body(W_ref, *refs):
    ins = refs[:6]
    a1h_ref, bh_ref, z_ref, h0, h1, o_ref = refs[6:]
    acc = _dots(W_ref, ins)
    ht = jnp.tanh(acc + a1h_ref[...] + bh_ref[...])
    z = z_ref[...]
    hcat = jnp.concatenate([h0[0], h1[0]], axis=1)
    o_ref[...] = z * hcat + (1.0 - z) * ht


def _hspec(h):
    return pl.BlockSpec((1, _BM, CH), lambda i, h=h: (h, i, 0))


def _stk_specs(n):
    return [_hspec(h) for _ in range(n) for h in (0, 1)]


def _mm1(Wxs, xs, l1x, l2x):
    return pl.pallas_call(
        _mm1_body,
        grid=(_GRID,),
        in_specs=[pl.BlockSpec((6, CH, 3 * C), lambda i: (0, 0, 0))]
        + _stk_specs(3),
        out_specs=pl.BlockSpec((_BM, 3 * C), lambda i: (i, 0)),
        out_shape=jax.ShapeDtypeStruct((NP, 3 * C), _f32),
    )(Wxs, xs, xs, l1x, l1x, l2x, l2x)


def _mm2(Whs, hs, l1h, l2h, A1, bzr):
    return pl.pallas_call(
        _mm2_body,
        grid=(_GRID,),
        in_specs=[pl.BlockSpec((6, CH, 2 * C), lambda i: (0, 0, 0))]
        + _stk_specs(3)
        + [pl.BlockSpec((_BM, 2 * C), lambda i: (i, 0)),
           pl.BlockSpec((1, 2 * C), lambda i: (0, 0))],
        out_specs=(pl.BlockSpec((_BM, C), lambda i: (i, 0)),
                   pl.BlockSpec((2, _BM, CH), lambda i: (0, i, 0))),
        out_shape=(jax.ShapeDtypeStruct((NP, C), _f32), _STK),
    )(Whs, hs, hs, l1h, l1h, l2h, l2h, A1, bzr)


def _mm3(Wgs, gs, l1g, l2g, A1, bh, Z, hs):
    return pl.pallas_call(
        _mm3_body,
        grid=(_GRID,),
        in_specs=[pl.BlockSpec((6, CH, C), lambda i: (0, 0, 0))]
        + _stk_specs(3)
        + [pl.BlockSpec((_BM, C), lambda i: (i, 2)),
           pl.BlockSpec((1, C), lambda i: (0, 0)),
           pl.BlockSpec((_BM, C), lambda i: (i, 0)),
           _hspec(0), _hspec(1)],
        out_specs=pl.BlockSpec((_BM, C), lambda i: (i, 0)),
        out_shape=jax.ShapeDtypeStruct((NP, C), _f32),
    )(Wgs, gs, gs, l1g, l1g, l2g, l2g, A1, bh, Z, hs, hs)


def _eff(W):
    return jnp.stack([W[0] - W[2], W[1], 2.0 * W[2]])


def _split6(Ws):
    return jnp.concatenate(
        [jnp.stack([Ws[k, :CH], Ws[k, CH:]]) for k in range(3)])


def _stack(A):
    return jnp.stack([A[:, :CH], A[:, CH:]])


def kernel(X, edge_index, edge_weight, H,
           W_xz, b_xz, W_hz, b_hz,
           W_xr, b_xr, W_hr, b_hr,
           W_xh, b_xh, W_hh, b_hh):
    row = edge_index[0].astype(_i32)
    col = edge_index[1].astype(_i32)
    rowTf = row.reshape(NT, 1, EPT)
    colTf = col.reshape(NT, 1, EPT)
    wTf = edge_weight.reshape(NT, 1, EPT)
    rowW = row.reshape(32, 1, EPW)
    wW = edge_weight.reshape(32, 1, EPW)

    pad = ((0, NP - N), (0, 0))
    Xs = _stack(jnp.pad(X, pad))
    Hs = _stack(jnp.pad(H, pad))
    zN = jnp.zeros((1, NP), _f32)
    zacc = jnp.zeros((NP, CH), _f32)

    Wxs = _split6(jnp.concatenate([_eff(W_xz), _eff(W_xr), _eff(W_xh)],
                                  axis=2))
    Whs = _split6(jnp.concatenate([_eff(W_hz), _eff(W_hr)], axis=2))
    Wgs = _split6(_eff(W_hh))
    bzr = jnp.concatenate([b_xz + b_hz, b_xr + b_hr])[None, :]
    bh = (b_xh + b_hh)[None, :]

    degP = _degpart(rowW, wW, zN)
    dis_r = _dis(degP.reshape(32, NP))
    normTf = _norm(rowTf, colTf, wTf, dis_r)

    L1x, L2x, L1h, L2h = _chainxh(Xs, Hs, rowTf, colTf, normTf, zacc)
    A1 = _mm1(Wxs, Xs, L1x, L2x)
    Z, Gs = _mm2(Whs, Hs, L1h, L2h, A1, bzr)

    L1g, L2g = _chaing(Gs, rowTf, colTf, normTf, zacc)
    out = _mm3(Wgs, Gs, L1g, L2g, A1, bh, Z, Hs)
    return out[:N]
